# baseline (speedup 1.0000x reference)
import functools

import jax
import jax.numpy as jnp
from jax.experimental import pallas as pl
from jax.experimental.pallas import tpu as pltpu

N = 10000
D = 128
K = 16
M = K + 1
REG = 0.05
ITERS = 3
BN = 200


def _sinkhorn_body(p_ref, cost_ref, out_ref):
    P = p_ref[...]
    Km = jnp.exp(cost_ref[...] * (-1.0 / REG))
    r = jnp.sum(Km, axis=1)
    inv_m = jnp.float32(1.0 / M)

    def bary(Ktu):
        Ktu = jnp.maximum(Ktu, 1e-30)
        logsum = jnp.sum(jnp.log(Ktu).reshape(BN, M, D), axis=1)
        b = jnp.exp(logsum * inv_m)
        b_rows = jnp.broadcast_to(b[:, None, :], (BN, M, D)).reshape(BN * M, D)
        return b, b_rows / Ktu

    u = P / r[None, :]
    b, v = bary(jnp.dot(u, Km, preferred_element_type=jnp.float32))
    for it in range(1, ITERS):
        Kv = jnp.maximum(jnp.dot(v, Km.T, preferred_element_type=jnp.float32), 1e-30)
        u = P / Kv
        Ktu = jnp.dot(u, Km, preferred_element_type=jnp.float32)
        if it < ITERS - 1:
            b, v = bary(Ktu)
        else:
            b, _ = bary(Ktu)
    out_ref[...] = b / jnp.maximum(jnp.sum(b, axis=1, keepdims=True), 1e-30)


@functools.partial(jax.jit, static_argnames=("interpret",))
def _sinkhorn_tc(p_flat, cost, interpret=False):
    grid = N // BN
    return pl.pallas_call(
        _sinkhorn_body,
        grid=(grid,),
        in_specs=[
            pl.BlockSpec((BN * M, D), lambda i: (i, 0)),
            pl.BlockSpec((D, D), lambda i: (0, 0)),
        ],
        out_specs=pl.BlockSpec((BN, D), lambda i: (i, 0)),
        out_shape=jax.ShapeDtypeStruct((N, D), jnp.float32),
        interpret=interpret,
    )(p_flat, cost)


def kernel(trans_X, costMatrix, neigh_idx, interpret=False):
    idx = neigh_idx.astype(jnp.int32)
    self_col = jnp.arange(N, dtype=jnp.int32)[:, None]
    idx_full = jnp.concatenate([idx, self_col], axis=1).reshape(-1)
    p_flat = trans_X[idx_full]
    return _sinkhorn_tc(p_flat, costMatrix, interpret=interpret)


# speedup vs baseline: 1.8465x; 1.8465x over previous
import functools

import jax
import jax.numpy as jnp
from jax.experimental import pallas as pl
from jax.experimental.pallas import tpu as pltpu

N = 10000
D = 128
K = 16
M = K + 1
REG = 0.05
ITERS = 3
BN = 200


def _sinkhorn_body(p_ref, cost_ref, out_ref):
    P = p_ref[...].reshape(M * BN, D)
    Km = jnp.exp(cost_ref[...] * (-1.0 / REG))
    r = jnp.sum(Km, axis=1)
    inv_m = jnp.float32(1.0 / M)

    def bary(Ktu):
        Ktu = jnp.maximum(Ktu, 1e-30)
        logsum = jnp.sum(jnp.log(Ktu).reshape(M, BN, D), axis=0)
        b = jnp.exp(logsum * inv_m)
        b_rows = jnp.broadcast_to(b[None, :, :], (M, BN, D)).reshape(M * BN, D)
        return b, b_rows / Ktu

    u = P / r[None, :]
    b, v = bary(jnp.dot(u, Km, preferred_element_type=jnp.float32))
    for it in range(1, ITERS):
        Kv = jnp.maximum(jnp.dot(v, Km.T, preferred_element_type=jnp.float32), 1e-30)
        u = P / Kv
        Ktu = jnp.dot(u, Km, preferred_element_type=jnp.float32)
        if it < ITERS - 1:
            b, v = bary(Ktu)
        else:
            b, _ = bary(Ktu)
    out_ref[...] = b / jnp.maximum(jnp.sum(b, axis=1, keepdims=True), 1e-30)


@functools.partial(jax.jit, static_argnames=("interpret",))
def _sinkhorn_tc(p_flat, cost, interpret=False):
    grid = N // BN
    return pl.pallas_call(
        _sinkhorn_body,
        grid=(grid,),
        in_specs=[
            pl.BlockSpec((M, BN, D), lambda i: (0, i, 0)),
            pl.BlockSpec((D, D), lambda i: (0, 0)),
        ],
        out_specs=pl.BlockSpec((BN, D), lambda i: (i, 0)),
        out_shape=jax.ShapeDtypeStruct((N, D), jnp.float32),
        interpret=interpret,
    )(p_flat, cost)


def kernel(trans_X, costMatrix, neigh_idx, interpret=False):
    idx = neigh_idx.astype(jnp.int32)
    self_row = jnp.arange(N, dtype=jnp.int32)[None, :]
    idx_full = jnp.concatenate([idx.T, self_row], axis=0).reshape(-1)
    p_mm = trans_X[idx_full].reshape(M, N, D)
    return _sinkhorn_tc(p_mm, costMatrix, interpret=interpret)


# speedup vs baseline: 2.0585x; 1.1148x over previous
import functools

import jax
import jax.numpy as jnp
from jax import lax
from jax.experimental import pallas as pl
from jax.experimental.pallas import tpu as pltpu
from jax.experimental.pallas import tpu_sc as plsc

N = 10000
D = 128
K = 16
M = K + 1
REG = 0.05
ITERS = 3
BN = 200

NC, NS = 2, 16
NW = NC * NS
NPAD = 10240
NTOT = M * NPAD
ROWS_W = NTOT // NW
CHUNK = 80
CH_W = ROWS_W // CHUNK


def _gather_body(x_hbm, idx_hbm, out_hbm, idx_v, buf0, buf1, sem0, sem1):
    wid = lax.axis_index("s") * NC + lax.axis_index("c")
    rbase = wid * ROWS_W
    pltpu.sync_copy(idx_hbm.at[wid], idx_v)
    bufs = (buf0, buf1)
    sems = (sem0, sem1)

    def g(j):
        return pltpu.make_async_copy(
            x_hbm.at[idx_v.at[j]], bufs[j % 2], sems[j % 2])

    g(0).start()
    for j in range(CH_W):
        if j + 1 < CH_W:
            g(j + 1).start()
        g(j).wait()
        pltpu.sync_copy(bufs[j % 2],
                        out_hbm.at[pl.ds(rbase + j * CHUNK, CHUNK)])


_sc_gather = functools.partial(
    pl.kernel,
    mesh=plsc.VectorSubcoreMesh(core_axis_name="c", subcore_axis_name="s"),
    out_type=jax.ShapeDtypeStruct((NTOT, D), jnp.float32),
    scratch_types=[
        pltpu.VMEM((CH_W, CHUNK), jnp.int32),
        pltpu.VMEM((CHUNK, D), jnp.float32),
        pltpu.VMEM((CHUNK, D), jnp.float32),
        pltpu.SemaphoreType.DMA,
        pltpu.SemaphoreType.DMA,
    ],
)(_gather_body)


def _sinkhorn_body(p_ref, cost_ref, out_ref):
    P = p_ref[...].reshape(M * BN, D)
    Km = jnp.exp(cost_ref[...] * (-1.0 / REG))
    r = jnp.sum(Km, axis=1)
    inv_m = jnp.float32(1.0 / M)

    def bary(Ktu):
        Ktu = jnp.maximum(Ktu, 1e-30)
        logsum = jnp.sum(jnp.log(Ktu).reshape(M, BN, D), axis=0)
        b = jnp.exp(logsum * inv_m)
        b_rows = jnp.broadcast_to(b[None, :, :], (M, BN, D)).reshape(M * BN, D)
        return b, b_rows / Ktu

    u = P / r[None, :]
    b, v = bary(jnp.dot(u, Km, preferred_element_type=jnp.float32))
    for it in range(1, ITERS):
        Kv = jnp.maximum(jnp.dot(v, Km.T, preferred_element_type=jnp.float32), 1e-30)
        u = P / Kv
        Ktu = jnp.dot(u, Km, preferred_element_type=jnp.float32)
        if it < ITERS - 1:
            b, v = bary(Ktu)
        else:
            b, _ = bary(Ktu)
    out_ref[...] = b / jnp.maximum(jnp.sum(b, axis=1, keepdims=True), 1e-30)


def _sinkhorn_tc(p_mm, cost):
    grid = N // BN
    return pl.pallas_call(
        _sinkhorn_body,
        grid=(grid,),
        in_specs=[
            pl.BlockSpec((M, BN, D), lambda i: (0, i, 0)),
            pl.BlockSpec((D, D), lambda i: (0, 0)),
        ],
        out_specs=pl.BlockSpec((BN, D), lambda i: (i, 0)),
        out_shape=jax.ShapeDtypeStruct((N, D), jnp.float32),
    )(p_mm, cost)


@jax.jit
def kernel(trans_X, costMatrix, neigh_idx):
    idx = neigh_idx.astype(jnp.int32)
    self_row = jnp.arange(N, dtype=jnp.int32)[None, :]
    idx_mm = jnp.concatenate([idx.T, self_row], axis=0)
    idx_mm = jnp.pad(idx_mm, ((0, 0), (0, NPAD - N)))
    idx3d = idx_mm.reshape(NW, CH_W, CHUNK)
    p_flat = _sc_gather(trans_X, idx3d)
    p_mm = p_flat.reshape(M, NPAD, D)
    return _sinkhorn_tc(p_mm, costMatrix)


# speedup vs baseline: 2.4585x; 1.1943x over previous
import functools

import jax
import jax.numpy as jnp
from jax import lax
from jax.experimental import pallas as pl
from jax.experimental.pallas import tpu as pltpu
from jax.experimental.pallas import tpu_sc as plsc

N = 10000
D = 128
K = 16
M = K + 1
REG = 0.05
ITERS = 3
BN = 200

NC, NS = 2, 16
NW = NC * NS
NPAD = 10240
NTOT = K * NPAD
ROWS_W = NTOT // NW
CHUNK = 80
CH_W = ROWS_W // CHUNK
NB = 4


def _gather_body(x_hbm, idx_hbm, out_hbm, idx_v, *scr):
    bufs = scr[:NB]
    gsems = scr[NB:2 * NB]
    ssems = scr[2 * NB:3 * NB]
    wid = lax.axis_index("s") * NC + lax.axis_index("c")
    rbase = wid * ROWS_W
    pltpu.sync_copy(idx_hbm.at[wid], idx_v)

    def g(j):
        return pltpu.make_async_copy(
            x_hbm.at[idx_v.at[j]], bufs[j % NB], gsems[j % NB])

    def s(j):
        return pltpu.make_async_copy(
            bufs[j % NB], out_hbm.at[pl.ds(rbase + j * CHUNK, CHUNK)],
            ssems[j % NB])

    for j in range(NB):
        g(j).start()
    for j in range(CH_W):
        g(j).wait()
        s(j).start()
        if j >= 1 and j - 1 + NB < CH_W:
            s(j - 1).wait()
            g(j - 1 + NB).start()
    for j in range(max(0, CH_W - NB), CH_W):
        s(j).wait()


@functools.cache
def _sc_gather():
    return pl.kernel(
        _gather_body,
        mesh=plsc.VectorSubcoreMesh(core_axis_name="c", subcore_axis_name="s"),
        out_type=jax.ShapeDtypeStruct((NTOT, D), jnp.float32),
        scratch_types=(
            [pltpu.VMEM((CH_W, CHUNK), jnp.int32)]
            + [pltpu.VMEM((CHUNK, D), jnp.float32) for _ in range(NB)]
            + [pltpu.SemaphoreType.DMA for _ in range(2 * NB)]
        ),
    )


def _sinkhorn_body(p_ref, x_ref, cost_ref, out_ref):
    P = jnp.concatenate([p_ref[...].reshape(K * BN, D), x_ref[...]], axis=0)
    Km = jnp.exp(cost_ref[...] * (-1.0 / REG))
    r = jnp.sum(Km, axis=1)
    inv_m = jnp.float32(1.0 / M)

    def bary(Ktu):
        Ktu = jnp.maximum(Ktu, 1e-30)
        logsum = jnp.sum(jnp.log(Ktu).reshape(M, BN, D), axis=0)
        b = jnp.exp(logsum * inv_m)
        b_rows = jnp.broadcast_to(b[None, :, :], (M, BN, D)).reshape(M * BN, D)
        return b, b_rows / Ktu

    Kms = Km * (1.0 / r)[:, None]
    b, v = bary(jnp.dot(P, Kms, preferred_element_type=jnp.float32))
    for it in range(1, ITERS):
        Kv = jnp.maximum(jnp.dot(v, Km.T, preferred_element_type=jnp.float32), 1e-30)
        u = P / Kv
        Ktu = jnp.dot(u, Km, preferred_element_type=jnp.float32)
        if it < ITERS - 1:
            b, v = bary(Ktu)
        else:
            b, _ = bary(Ktu)
    out_ref[...] = b / jnp.maximum(jnp.sum(b, axis=1, keepdims=True), 1e-30)


def _sinkhorn_tc(p_mm, x, cost):
    grid = N // BN
    return pl.pallas_call(
        _sinkhorn_body,
        grid=(grid,),
        in_specs=[
            pl.BlockSpec((K, BN, D), lambda i: (0, i, 0)),
            pl.BlockSpec((BN, D), lambda i: (i, 0)),
            pl.BlockSpec((D, D), lambda i: (0, 0)),
        ],
        out_specs=pl.BlockSpec((BN, D), lambda i: (i, 0)),
        out_shape=jax.ShapeDtypeStruct((N, D), jnp.float32),
    )(p_mm, x, cost)


@jax.jit
def kernel(trans_X, costMatrix, neigh_idx):
    idx = neigh_idx.astype(jnp.int32)
    idx_mm = jnp.pad(idx.T, ((0, 0), (0, NPAD - N)))
    idx3d = idx_mm.reshape(NW, CH_W, CHUNK)
    p_flat = _sc_gather()(trans_X, idx3d)
    p_mm = p_flat.reshape(K, NPAD, D)
    return _sinkhorn_tc(p_mm, trans_X, costMatrix)
